# baseline (speedup 1.0000x reference)
import functools

import jax
import jax.numpy as jnp
from jax import lax
from jax.experimental import pallas as pl
from jax.experimental.pallas import tpu as pltpu
from jax.experimental.pallas import tpu_sc as plsc

_K = 40
_N = 2048
_C = 512
_O = 515
_NH = 8
_AD = 64
_SCALE = (_NH * _AD) ** -0.5
_BNS = 1.0 / (1.0 + 1e-5) ** 0.5

_RT = 256

_ROWS = _K * _N
_NW = 32
_PW = _ROWS // _NW
_CR = 128
_NCH = _PW // _CR


def _knn_body(s_blk_ref, s_all_ref, idx_ref):
    s_blk = s_blk_ref[...]
    s_all = s_all_ref[...]
    g = lax.dot_general(s_blk, s_all, (((1,), (1,)), ((), ())))
    xx_blk = jnp.sum(s_blk * s_blk, axis=1)[:, None]
    xx_all = jnp.sum(s_all * s_all, axis=1)[None, :]
    pd = 2.0 * g - xx_blk - xx_all
    iob = lax.broadcasted_iota(jnp.int32, (_RT, 128), 1)
    iotas = [iob + i * 128 for i in range(_N // 128)]
    neg = jnp.full((_RT, 128), -jnp.inf, jnp.float32)
    big = jnp.full((_RT, 128), _N, jnp.int32)

    def body(j, vprev):
        vp = vprev[:, None]
        mx = neg
        am = big
        for i in range(_N // 128):
            blk = pd[:, i * 128:(i + 1) * 128]
            mx = jnp.maximum(mx, jnp.where(blk < vp, blk, neg))
            am = jnp.minimum(am, jnp.where(blk == vp, iotas[i], big))
        m = jnp.max(mx, axis=1)
        arg = jnp.min(am, axis=1)

        @pl.when(j > 0)
        def _():
            idx_ref[pl.ds(j - 1, 1), :] = arg[None, :]

        return m

    lax.fori_loop(0, _K + 1, body, jnp.full((_RT,), jnp.inf, jnp.float32))


def _make_knn(interpret=False):
    return pl.pallas_call(
        _knn_body,
        grid=(_N // _RT,),
        in_specs=[
            pl.BlockSpec((_RT, _C), lambda r: (r, 0)),
            pl.BlockSpec((_N, _C), lambda r: (0, 0)),
        ],
        out_specs=pl.BlockSpec((_K, _RT), lambda r: (0, r)),
        out_shape=jax.ShapeDtypeStruct((_K, _N), jnp.int32),
        interpret=interpret,
    )


def _uv_body(s_ref, w5_ref, g5_ref, b5_ref, u_ref, v_ref):
    s = s_ref[...]
    wa = w5_ref[:, :_C]
    wd = w5_ref[:, _C:] - wa
    u = lax.dot_general(s, wa, (((1,), (1,)), ((), ())))
    v = lax.dot_general(s, wd, (((1,), (1,)), ((), ())))
    s5 = (g5_ref[0] * _BNS)[None, :]
    us = u * s5
    ub = lax.bitcast_convert_type(us, jnp.int32)
    ub = (ub + 0x7FFF + ((ub >> 16) & 1)) >> 16
    lo = ub[:, : _C // 2] & 0xFFFF
    hi = ub[:, _C // 2:] << 16
    u_ref[...] = lo | hi
    v_ref[...] = v * s5 + b5_ref[0][None, :]


def _make_uv(interpret=False):
    return pl.pallas_call(
        _uv_body,
        in_specs=[
            pl.BlockSpec((_N, _C), lambda: (0, 0)),
            pl.BlockSpec((_C, 2 * _C), lambda: (0, 0)),
            pl.BlockSpec((1, _C), lambda: (0, 0)),
            pl.BlockSpec((1, _C), lambda: (0, 0)),
        ],
        out_specs=[
            pl.BlockSpec((_N, _C // 2), lambda: (0, 0)),
            pl.BlockSpec((_N, _C), lambda: (0, 0)),
        ],
        out_shape=[
            jax.ShapeDtypeStruct((_N, _C // 2), jnp.int32),
            jax.ShapeDtypeStruct((_N, _C), jnp.float32),
        ],
        interpret=interpret,
    )


def _gather_body(tab_ref, idx_ref, out_ref, iv0, rv0, iv1, rv1, sem0, sem1):
    wid = lax.axis_index("s") * 2 + lax.axis_index("c")
    base = wid * _PW

    def fire(iv, rv, sem, off):
        pltpu.sync_copy(idx_ref.at[pl.ds(off, _CR)], iv)
        pltpu.make_async_copy(tab_ref.at[iv], rv, sem).start()

    def drain(iv, rv, sem, off):
        pltpu.make_async_copy(tab_ref.at[iv], rv, sem).wait()
        pltpu.sync_copy(rv, out_ref.at[pl.ds(off, _CR)])

    fire(iv0, rv0, sem0, base)

    def pair(i, carry):
        o0 = base + (2 * i) * _CR
        o1 = o0 + _CR
        fire(iv1, rv1, sem1, o1)
        drain(iv0, rv0, sem0, o0)

        @pl.when(i < _NCH // 2 - 1)
        def _():
            fire(iv0, rv0, sem0, o1 + _CR)

        drain(iv1, rv1, sem1, o1)
        return carry

    lax.fori_loop(0, _NCH // 2, pair, 0)


@functools.lru_cache(maxsize=None)
def _make_sc_gather():
    return pl.kernel(
        _gather_body,
        out_type=jax.ShapeDtypeStruct((_ROWS, _C // 2), jnp.int32),
        mesh=plsc.VectorSubcoreMesh(core_axis_name="c", subcore_axis_name="s"),
        scratch_types=[
            pltpu.VMEM((_CR,), jnp.int32),
            pltpu.VMEM((_CR, _C // 2), jnp.int32),
            pltpu.VMEM((_CR,), jnp.int32),
            pltpu.VMEM((_CR, _C // 2), jnp.int32),
            pltpu.SemaphoreType.DMA,
            pltpu.SemaphoreType.DMA,
        ],
    )


def _conv2_body(g_ref, v_ref, w6_ref, g6_ref, b6_ref, out_ref):
    j = pl.program_id(0)
    gp = g_ref[...]
    g_lo = lax.bitcast_convert_type(gp << 16, jnp.float32)
    g_hi = lax.bitcast_convert_type(gp & jnp.int32(-65536), jnp.float32)
    h = jnp.concatenate([g_lo, g_hi], axis=1) + v_ref[...]
    h = jnp.maximum(h, 0.2 * h)
    f = lax.dot_general(w6_ref[...], h, (((1,), (0,)), ((), ())))
    a = f * (g6_ref[0] * _BNS)[:, None] + b6_ref[0][:, None]
    a = jnp.maximum(a, 0.2 * a)

    @pl.when(j == 0)
    def _():
        out_ref[...] = a

    @pl.when(j > 0)
    def _():
        out_ref[...] = jnp.maximum(out_ref[...], a)


def _make_conv2(interpret=False):
    return pl.pallas_call(
        _conv2_body,
        grid=(_K,),
        in_specs=[
            pl.BlockSpec((_N, _C // 2), lambda j: (j, 0)),
            pl.BlockSpec((_N, _C), lambda j: (0, 0)),
            pl.BlockSpec((_O, _N), lambda j: (0, 0)),
            pl.BlockSpec((1, _O), lambda j: (0, 0)),
            pl.BlockSpec((1, _O), lambda j: (0, 0)),
        ],
        out_specs=pl.BlockSpec((_O, _C), lambda j: (0, 0)),
        out_shape=jax.ShapeDtypeStruct((_O, _C), jnp.float32),
        interpret=interpret,
    )


def _attn_body(xf_ref, yf_ref, wq_ref, wk_ref, wv_ref, ow_ref, ob_ref, out_ref):
    xf = xf_ref[...]
    yf = yf_ref[...]
    qt = lax.dot_general(wq_ref[...], xf, (((1,), (0,)), ((), ())))
    kt = lax.dot_general(wk_ref[...], yf, (((1,), (0,)), ((), ())))
    vt = lax.dot_general(wv_ref[...], yf, (((1,), (0,)), ((), ())))
    acc = jnp.zeros((_C, _O), jnp.float32)
    for h in range(_NH):
        sl = slice(h * _AD, (h + 1) * _AD)
        qh = qt[sl, :]
        kh = kt[sl, :]
        vh = vt[sl, :]
        dp = lax.dot_general(qh, kh, (((0,), (0,)), ((), ()))) * _SCALE
        m = jnp.max(dp, axis=1, keepdims=True)
        e = jnp.exp(dp - m)
        p = e / jnp.sum(e, axis=1, keepdims=True)
        wh = lax.dot_general(p, vh, (((1,), (1,)), ((), ())))
        owh = ow_ref[:, sl]
        acc = acc + lax.dot_general(wh, owh, (((1,), (1,)), ((), ())))
    out_ref[...] = acc + ob_ref[0][None, :]


def _make_attn(interpret=False):
    return pl.pallas_call(
        _attn_body,
        in_specs=[
            pl.BlockSpec((_O, _C), lambda: (0, 0)),
            pl.BlockSpec((_O, _C), lambda: (0, 0)),
            pl.BlockSpec((_NH * _AD, _O), lambda: (0, 0)),
            pl.BlockSpec((_NH * _AD, _O), lambda: (0, 0)),
            pl.BlockSpec((_NH * _AD, _O), lambda: (0, 0)),
            pl.BlockSpec((_O, _NH * _AD), lambda: (0, 0)),
            pl.BlockSpec((1, _O), lambda: (0, 0)),
        ],
        out_specs=pl.BlockSpec((_C, _O), lambda: (0, 0)),
        out_shape=jax.ShapeDtypeStruct((_C, _O), jnp.float32),
        interpret=interpret,
    )


def kernel(x, y, w5, g5, b5, w6, g6, b6, wq, wk, wv, ow, ob):
    sx = x[0].T
    sy = y[0].T
    g5r = g5.reshape(1, _C)
    b5r = b5.reshape(1, _C)
    g6r = g6.reshape(1, _O)
    b6r = b6.reshape(1, _O)
    obr = ob.reshape(1, _O)

    knn = _make_knn()
    uv = _make_uv()
    conv2 = _make_conv2()
    gather = _make_sc_gather()

    idx_x = knn(sx, sx)
    ux, vx = uv(sx, w5, g5r, b5r)
    gx = gather(ux, idx_x.reshape(_ROWS))
    idx_y = knn(sy, sy)
    uy, vy = uv(sy, w5, g5r, b5r)
    gy = gather(uy, idx_y.reshape(_ROWS))
    xft = conv2(gx, vx, w6, g6r, b6r)
    yft = conv2(gy, vy, w6, g6r, b6r)
    out = _make_attn()(xft, yft, wq, wk, wv, ow, obr)
    return out[None]


# speedup vs baseline: 1.0460x; 1.0460x over previous
import functools

import jax
import jax.numpy as jnp
from jax import lax
from jax.experimental import pallas as pl
from jax.experimental.pallas import tpu as pltpu
from jax.experimental.pallas import tpu_sc as plsc

_K = 40
_N = 2048
_C = 512
_O = 515
_NH = 8
_AD = 64
_SCALE = (_NH * _AD) ** -0.5
_BNS = 1.0 / (1.0 + 1e-5) ** 0.5

_RT = 256

_ROWS = _K * _N
_NW = 32
_PW = _ROWS // _NW
_CR = 128
_NCH = _PW // _CR


def _knn_body(s_blk_ref, s_all_ref, idx_ref):
    s_blk = s_blk_ref[...]
    s_all = s_all_ref[...]
    g = lax.dot_general(s_blk, s_all, (((0,), (0,)), ((), ())))
    xx_blk = jnp.sum(s_blk * s_blk, axis=0)[:, None]
    xx_all = jnp.sum(s_all * s_all, axis=0)[None, :]
    pd = 2.0 * g - xx_blk - xx_all
    iob = lax.broadcasted_iota(jnp.int32, (_RT, 128), 1)
    iotas = [iob + i * 128 for i in range(_N // 128)]
    neg = jnp.full((_RT, 128), -jnp.inf, jnp.float32)
    big = jnp.full((_RT, 128), _N, jnp.int32)

    def body(j, vprev):
        vp = vprev[:, None]
        mx = neg
        am = big
        for i in range(_N // 128):
            blk = pd[:, i * 128:(i + 1) * 128]
            mx = jnp.maximum(mx, jnp.where(blk < vp, blk, neg))
            am = jnp.minimum(am, jnp.where(blk == vp, iotas[i], big))
        m = jnp.max(mx, axis=1)
        arg = jnp.min(am, axis=1)

        @pl.when(j > 0)
        def _():
            idx_ref[pl.ds(j - 1, 1), :] = arg[None, :]

        return m

    lax.fori_loop(0, _K + 1, body, jnp.full((_RT,), jnp.inf, jnp.float32))


def _make_knn(interpret=False):
    return pl.pallas_call(
        _knn_body,
        grid=(_N // _RT,),
        in_specs=[
            pl.BlockSpec((_C, _RT), lambda r: (0, r)),
            pl.BlockSpec((_C, _N), lambda r: (0, 0)),
        ],
        out_specs=pl.BlockSpec((_K, _RT), lambda r: (0, r)),
        out_shape=jax.ShapeDtypeStruct((_K, _N), jnp.int32),
        interpret=interpret,
    )


def _uv_body(s_ref, w5_ref, g5_ref, b5_ref, u_ref, v_ref):
    s = s_ref[...]
    wa = w5_ref[:, :_C]
    wd = w5_ref[:, _C:] - wa
    u = lax.dot_general(s, wa, (((0,), (1,)), ((), ())))
    v = lax.dot_general(s, wd, (((0,), (1,)), ((), ())))
    s5 = (g5_ref[0] * _BNS)[None, :]
    us = u * s5
    ub = lax.bitcast_convert_type(us, jnp.int32)
    ub = (ub + 0x7FFF + ((ub >> 16) & 1)) >> 16
    lo = ub[:, : _C // 2] & 0xFFFF
    hi = ub[:, _C // 2:] << 16
    u_ref[...] = lo | hi
    v_ref[...] = v * s5 + b5_ref[0][None, :]


def _make_uv(interpret=False):
    return pl.pallas_call(
        _uv_body,
        in_specs=[
            pl.BlockSpec((_C, _N), lambda: (0, 0)),
            pl.BlockSpec((_C, 2 * _C), lambda: (0, 0)),
            pl.BlockSpec((1, _C), lambda: (0, 0)),
            pl.BlockSpec((1, _C), lambda: (0, 0)),
        ],
        out_specs=[
            pl.BlockSpec((_N, _C // 2), lambda: (0, 0)),
            pl.BlockSpec((_N, _C), lambda: (0, 0)),
        ],
        out_shape=[
            jax.ShapeDtypeStruct((_N, _C // 2), jnp.int32),
            jax.ShapeDtypeStruct((_N, _C), jnp.float32),
        ],
        interpret=interpret,
    )


def _gather_body(tab_ref, idx_ref, out_ref, idx_v, rows_v, sem):
    wid = lax.axis_index("s") * 2 + lax.axis_index("c")
    base = wid * _PW

    def chunk(i, carry):
        off = base + i * _CR
        pltpu.sync_copy(idx_ref.at[pl.ds(off, _CR)], idx_v)
        pltpu.async_copy(tab_ref.at[idx_v], rows_v, sem).wait()
        pltpu.sync_copy(rows_v, out_ref.at[pl.ds(off, _CR)])
        return carry

    lax.fori_loop(0, _NCH, chunk, 0)


@functools.lru_cache(maxsize=None)
def _make_sc_gather():
    return pl.kernel(
        _gather_body,
        out_type=jax.ShapeDtypeStruct((_ROWS, _C // 2), jnp.int32),
        mesh=plsc.VectorSubcoreMesh(core_axis_name="c", subcore_axis_name="s"),
        scratch_types=[
            pltpu.VMEM((_CR,), jnp.int32),
            pltpu.VMEM((_CR, _C // 2), jnp.int32),
            pltpu.SemaphoreType.DMA,
        ],
    )


def _conv2_body(g_ref, v_ref, w6_ref, g6_ref, b6_ref, out_ref):
    j = pl.program_id(0)
    gp = g_ref[...]
    g_lo = lax.bitcast_convert_type(gp << 16, jnp.float32)
    g_hi = lax.bitcast_convert_type(gp & jnp.int32(-65536), jnp.float32)
    h = jnp.concatenate([g_lo, g_hi], axis=1) + v_ref[...]
    h = jnp.maximum(h, 0.2 * h)
    f = lax.dot_general(w6_ref[...], h, (((1,), (0,)), ((), ())))
    a = f * (g6_ref[0] * _BNS)[:, None] + b6_ref[0][:, None]
    a = jnp.maximum(a, 0.2 * a)

    @pl.when(j == 0)
    def _():
        out_ref[...] = a

    @pl.when(j > 0)
    def _():
        out_ref[...] = jnp.maximum(out_ref[...], a)


def _make_conv2(interpret=False):
    return pl.pallas_call(
        _conv2_body,
        grid=(_K,),
        in_specs=[
            pl.BlockSpec((_N, _C // 2), lambda j: (j, 0)),
            pl.BlockSpec((_N, _C), lambda j: (0, 0)),
            pl.BlockSpec((_O, _N), lambda j: (0, 0)),
            pl.BlockSpec((1, _O), lambda j: (0, 0)),
            pl.BlockSpec((1, _O), lambda j: (0, 0)),
        ],
        out_specs=pl.BlockSpec((_O, _C), lambda j: (0, 0)),
        out_shape=jax.ShapeDtypeStruct((_O, _C), jnp.float32),
        interpret=interpret,
    )


def _attn_body(xf_ref, yf_ref, wq_ref, wk_ref, wv_ref, ow_ref, ob_ref, out_ref):
    xf = xf_ref[...]
    yf = yf_ref[...]
    qt = lax.dot_general(wq_ref[...], xf, (((1,), (0,)), ((), ())))
    kt = lax.dot_general(wk_ref[...], yf, (((1,), (0,)), ((), ())))
    vt = lax.dot_general(wv_ref[...], yf, (((1,), (0,)), ((), ())))
    acc = jnp.zeros((_C, _O), jnp.float32)
    for h in range(_NH):
        sl = slice(h * _AD, (h + 1) * _AD)
        qh = qt[sl, :]
        kh = kt[sl, :]
        vh = vt[sl, :]
        dp = lax.dot_general(qh, kh, (((0,), (0,)), ((), ()))) * _SCALE
        m = jnp.max(dp, axis=1, keepdims=True)
        e = jnp.exp(dp - m)
        p = e / jnp.sum(e, axis=1, keepdims=True)
        wh = lax.dot_general(p, vh, (((1,), (1,)), ((), ())))
        owh = ow_ref[:, sl]
        acc = acc + lax.dot_general(wh, owh, (((1,), (1,)), ((), ())))
    out_ref[...] = acc + ob_ref[0][None, :]


def _make_attn(interpret=False):
    return pl.pallas_call(
        _attn_body,
        in_specs=[
            pl.BlockSpec((_O, _C), lambda: (0, 0)),
            pl.BlockSpec((_O, _C), lambda: (0, 0)),
            pl.BlockSpec((_NH * _AD, _O), lambda: (0, 0)),
            pl.BlockSpec((_NH * _AD, _O), lambda: (0, 0)),
            pl.BlockSpec((_NH * _AD, _O), lambda: (0, 0)),
            pl.BlockSpec((_O, _NH * _AD), lambda: (0, 0)),
            pl.BlockSpec((1, _O), lambda: (0, 0)),
        ],
        out_specs=pl.BlockSpec((_C, _O), lambda: (0, 0)),
        out_shape=jax.ShapeDtypeStruct((_C, _O), jnp.float32),
        interpret=interpret,
    )


def kernel(x, y, w5, g5, b5, w6, g6, b6, wq, wk, wv, ow, ob):
    sx = x[0]
    sy = y[0]
    g5r = g5.reshape(1, _C)
    b5r = b5.reshape(1, _C)
    g6r = g6.reshape(1, _O)
    b6r = b6.reshape(1, _O)
    obr = ob.reshape(1, _O)

    knn = _make_knn()
    uv = _make_uv()
    conv2 = _make_conv2()
    gather = _make_sc_gather()

    idx_x = knn(sx, sx)
    ux, vx = uv(sx, w5, g5r, b5r)
    gx = gather(ux, idx_x.reshape(_ROWS))
    idx_y = knn(sy, sy)
    uy, vy = uv(sy, w5, g5r, b5r)
    gy = gather(uy, idx_y.reshape(_ROWS))
    xft = conv2(gx, vx, w6, g6r, b6r)
    yft = conv2(gy, vy, w6, g6r, b6r)
    out = _make_attn()(xft, yft, wq, wk, wv, ow, obr)
    return out[None]


# speedup vs baseline: 1.0857x; 1.0379x over previous
import functools

import jax
import jax.numpy as jnp
from jax import lax
from jax.experimental import pallas as pl
from jax.experimental.pallas import tpu as pltpu
from jax.experimental.pallas import tpu_sc as plsc

_K = 40
_N = 2048
_C = 512
_O = 515
_NH = 8
_AD = 64
_SCALE = (_NH * _AD) ** -0.5
_BNS = 1.0 / (1.0 + 1e-5) ** 0.5

_RT = 256

_ROWS = _K * _N // 2
_NW = 32
_PW = _ROWS // _NW
_CR = 128
_NCH = _PW // _CR


def _knn_body(s_blk_ref, s_all_ref, idx_ref):
    s_blk = s_blk_ref[...]
    s_all = s_all_ref[...]
    g = lax.dot_general(s_blk, s_all, (((0,), (0,)), ((), ())))
    xx_blk = jnp.sum(s_blk * s_blk, axis=0)[:, None]
    xx_all = jnp.sum(s_all * s_all, axis=0)[None, :]
    pd = 2.0 * g - xx_blk - xx_all
    iob = lax.broadcasted_iota(jnp.int32, (_RT, 128), 1)
    iotas = [iob + i * 128 for i in range(_N // 128)]
    neg = jnp.full((_RT, 128), -jnp.inf, jnp.float32)
    big = jnp.full((_RT, 128), _N, jnp.int32)

    def body(j, vprev):
        vp = vprev[:, None]
        mx = neg
        am = big
        for i in range(_N // 128):
            blk = pd[:, i * 128:(i + 1) * 128]
            mx = jnp.maximum(mx, jnp.where(blk < vp, blk, neg))
            am = jnp.minimum(am, jnp.where(blk == vp, iotas[i], big))
        m = jnp.max(mx, axis=1)
        arg = jnp.min(am, axis=1)

        @pl.when(j > 0)
        def _():
            idx_ref[pl.ds(j - 1, 1), :] = arg[None, :]

        return m

    lax.fori_loop(0, _K + 1, body, jnp.full((_RT,), jnp.inf, jnp.float32))


def _make_knn(half, interpret=False):
    hb = half * (_N // 2) // _RT
    return pl.pallas_call(
        _knn_body,
        grid=(_N // 2 // _RT,),
        in_specs=[
            pl.BlockSpec((_C, _RT), lambda r: (0, hb + r)),
            pl.BlockSpec((_C, _N), lambda r: (0, 0)),
        ],
        out_specs=pl.BlockSpec((_K, _RT), lambda r: (0, r)),
        out_shape=jax.ShapeDtypeStruct((_K, _N // 2), jnp.int32),
        interpret=interpret,
    )


def _uv_body(s_ref, w5_ref, g5_ref, b5_ref, u_ref, v_ref):
    s = s_ref[...]
    wa = w5_ref[:, :_C]
    wd = w5_ref[:, _C:] - wa
    u = lax.dot_general(s, wa, (((0,), (1,)), ((), ())))
    v = lax.dot_general(s, wd, (((0,), (1,)), ((), ())))
    s5 = (g5_ref[0] * _BNS)[None, :]
    us = u * s5
    ub = lax.bitcast_convert_type(us, jnp.int32)
    ub = (ub + 0x7FFF + ((ub >> 16) & 1)) >> 16
    lo = ub[:, : _C // 2] & 0xFFFF
    hi = ub[:, _C // 2:] << 16
    u_ref[...] = lo | hi
    v_ref[...] = v * s5 + b5_ref[0][None, :]


def _make_uv(interpret=False):
    return pl.pallas_call(
        _uv_body,
        in_specs=[
            pl.BlockSpec((_C, _N), lambda: (0, 0)),
            pl.BlockSpec((_C, 2 * _C), lambda: (0, 0)),
            pl.BlockSpec((1, _C), lambda: (0, 0)),
            pl.BlockSpec((1, _C), lambda: (0, 0)),
        ],
        out_specs=[
            pl.BlockSpec((_N, _C // 2), lambda: (0, 0)),
            pl.BlockSpec((_N, _C), lambda: (0, 0)),
        ],
        out_shape=[
            jax.ShapeDtypeStruct((_N, _C // 2), jnp.int32),
            jax.ShapeDtypeStruct((_N, _C), jnp.float32),
        ],
        interpret=interpret,
    )


def _gather_body(tab_ref, idx_ref, out_ref, idx_v, rows_v, sem):
    wid = lax.axis_index("s") * 2 + lax.axis_index("c")
    base = wid * _PW

    def chunk(i, carry):
        off = base + i * _CR
        pltpu.sync_copy(idx_ref.at[pl.ds(off, _CR)], idx_v)
        pltpu.async_copy(tab_ref.at[idx_v], rows_v, sem).wait()
        pltpu.sync_copy(rows_v, out_ref.at[pl.ds(off, _CR)])
        return carry

    lax.fori_loop(0, _NCH, chunk, 0)


@functools.lru_cache(maxsize=None)
def _make_sc_gather():
    return pl.kernel(
        _gather_body,
        out_type=jax.ShapeDtypeStruct((_ROWS, _C // 2), jnp.int32),
        mesh=plsc.VectorSubcoreMesh(core_axis_name="c", subcore_axis_name="s"),
        scratch_types=[
            pltpu.VMEM((_CR,), jnp.int32),
            pltpu.VMEM((_CR, _C // 2), jnp.int32),
            pltpu.SemaphoreType.DMA,
        ],
    )


def _conv2_body(ga_ref, gb_ref, v_ref, w6_ref, g6_ref, b6_ref, out_ref):
    j = pl.program_id(0)
    gp = jnp.concatenate([ga_ref[...], gb_ref[...]], axis=0)
    g_lo = lax.bitcast_convert_type(gp << 16, jnp.float32)
    g_hi = lax.bitcast_convert_type(gp & jnp.int32(-65536), jnp.float32)
    h = jnp.concatenate([g_lo, g_hi], axis=1) + v_ref[...]
    h = jnp.maximum(h, 0.2 * h)
    f = lax.dot_general(w6_ref[...], h, (((1,), (0,)), ((), ())))
    a = f * (g6_ref[0] * _BNS)[:, None] + b6_ref[0][:, None]
    a = jnp.maximum(a, 0.2 * a)

    @pl.when(j == 0)
    def _():
        out_ref[...] = a

    @pl.when(j > 0)
    def _():
        out_ref[...] = jnp.maximum(out_ref[...], a)


def _make_conv2(interpret=False):
    return pl.pallas_call(
        _conv2_body,
        grid=(_K,),
        in_specs=[
            pl.BlockSpec((_N // 2, _C // 2), lambda j: (j, 0)),
            pl.BlockSpec((_N // 2, _C // 2), lambda j: (j, 0)),
            pl.BlockSpec((_N, _C), lambda j: (0, 0)),
            pl.BlockSpec((_O, _N), lambda j: (0, 0)),
            pl.BlockSpec((1, _O), lambda j: (0, 0)),
            pl.BlockSpec((1, _O), lambda j: (0, 0)),
        ],
        out_specs=pl.BlockSpec((_O, _C), lambda j: (0, 0)),
        out_shape=jax.ShapeDtypeStruct((_O, _C), jnp.float32),
        interpret=interpret,
    )


def _attn_body(xf_ref, yf_ref, wq_ref, wk_ref, wv_ref, ow_ref, ob_ref, out_ref):
    xf = xf_ref[...]
    yf = yf_ref[...]
    qt = lax.dot_general(wq_ref[...], xf, (((1,), (0,)), ((), ())))
    kt = lax.dot_general(wk_ref[...], yf, (((1,), (0,)), ((), ())))
    vt = lax.dot_general(wv_ref[...], yf, (((1,), (0,)), ((), ())))
    acc = jnp.zeros((_C, _O), jnp.float32)
    for h in range(_NH):
        sl = slice(h * _AD, (h + 1) * _AD)
        qh = qt[sl, :]
        kh = kt[sl, :]
        vh = vt[sl, :]
        dp = lax.dot_general(qh, kh, (((0,), (0,)), ((), ()))) * _SCALE
        m = jnp.max(dp, axis=1, keepdims=True)
        e = jnp.exp(dp - m)
        p = e / jnp.sum(e, axis=1, keepdims=True)
        wh = lax.dot_general(p, vh, (((1,), (1,)), ((), ())))
        owh = ow_ref[:, sl]
        acc = acc + lax.dot_general(wh, owh, (((1,), (1,)), ((), ())))
    out_ref[...] = acc + ob_ref[0][None, :]


def _make_attn(interpret=False):
    return pl.pallas_call(
        _attn_body,
        in_specs=[
            pl.BlockSpec((_O, _C), lambda: (0, 0)),
            pl.BlockSpec((_O, _C), lambda: (0, 0)),
            pl.BlockSpec((_NH * _AD, _O), lambda: (0, 0)),
            pl.BlockSpec((_NH * _AD, _O), lambda: (0, 0)),
            pl.BlockSpec((_NH * _AD, _O), lambda: (0, 0)),
            pl.BlockSpec((_O, _NH * _AD), lambda: (0, 0)),
            pl.BlockSpec((1, _O), lambda: (0, 0)),
        ],
        out_specs=pl.BlockSpec((_C, _O), lambda: (0, 0)),
        out_shape=jax.ShapeDtypeStruct((_C, _O), jnp.float32),
        interpret=interpret,
    )


def kernel(x, y, w5, g5, b5, w6, g6, b6, wq, wk, wv, ow, ob):
    sx = x[0]
    sy = y[0]
    g5r = g5.reshape(1, _C)
    b5r = b5.reshape(1, _C)
    g6r = g6.reshape(1, _O)
    b6r = b6.reshape(1, _O)
    obr = ob.reshape(1, _O)

    knn0 = _make_knn(0)
    knn1 = _make_knn(1)
    uv = _make_uv()
    conv2 = _make_conv2()
    gather = _make_sc_gather()

    ux, vx = uv(sx, w5, g5r, b5r)
    idx_xa = knn0(sx, sx)
    gxa = gather(ux, idx_xa.reshape(_ROWS))
    idx_xb = knn1(sx, sx)
    gxb = gather(ux, idx_xb.reshape(_ROWS))
    uy, vy = uv(sy, w5, g5r, b5r)
    idx_ya = knn0(sy, sy)
    gya = gather(uy, idx_ya.reshape(_ROWS))
    idx_yb = knn1(sy, sy)
    gyb = gather(uy, idx_yb.reshape(_ROWS))
    xft = conv2(gxa, gxb, vx, w6, g6r, b6r)
    yft = conv2(gya, gyb, vy, w6, g6r, b6r)
    out = _make_attn()(xft, yft, wq, wk, wv, ow, obr)
    return out[None]


# speedup vs baseline: 1.1716x; 1.0792x over previous
import functools

import jax
import jax.numpy as jnp
from jax import lax
from jax.experimental import pallas as pl
from jax.experimental.pallas import tpu as pltpu
from jax.experimental.pallas import tpu_sc as plsc

_K = 40
_N = 2048
_C = 512
_O = 515
_NH = 8
_AD = 64
_SCALE = (_NH * _AD) ** -0.5
_BNS = 1.0 / (1.0 + 1e-5) ** 0.5

_RT = 256

_ROWS = _K * _N // 2
_NW = 32
_PW = _ROWS // _NW
_CR = 128
_NCH = _PW // _CR


def _knn_body(s_blk_ref, s_all_ref, idx_ref):
    s_blk = s_blk_ref[...]
    s_all = s_all_ref[...]
    g = lax.dot_general(s_blk, s_all, (((0,), (0,)), ((), ())))
    xx_blk = jnp.sum(s_blk * s_blk, axis=0)[:, None]
    xx_all = jnp.sum(s_all * s_all, axis=0)[None, :]
    pd = 2.0 * g - xx_blk - xx_all
    iob = lax.broadcasted_iota(jnp.int32, (_RT, 128), 1)
    neg = jnp.full((_RT, 128), -jnp.inf, jnp.float32)
    big = jnp.full((_RT, 128), _N, jnp.int32)

    def body(j, vprev):
        vp = vprev[:, None]
        mx = neg
        bid = jnp.zeros((_RT, 128), jnp.int32)
        for i in range(_N // 128):
            blk = pd[:, i * 128:(i + 1) * 128]
            nm = jnp.where(blk < vp, blk, neg)
            upd = nm > mx
            mx = jnp.where(upd, nm, mx)
            bid = jnp.where(upd, i, bid)
        m = jnp.max(mx, axis=1)
        gi = bid * 128 + iob
        arg = jnp.min(jnp.where(mx == m[:, None], gi, big), axis=1)
        idx_ref[pl.ds(j, 1), :] = arg[None, :]
        return m

    lax.fori_loop(0, _K, body, jnp.full((_RT,), jnp.inf, jnp.float32))


def _make_knn(half, interpret=False):
    hb = half * (_N // 2) // _RT
    return pl.pallas_call(
        _knn_body,
        grid=(_N // 2 // _RT,),
        in_specs=[
            pl.BlockSpec((_C, _RT), lambda r: (0, hb + r)),
            pl.BlockSpec((_C, _N), lambda r: (0, 0)),
        ],
        out_specs=pl.BlockSpec((_K, _RT), lambda r: (0, r)),
        out_shape=jax.ShapeDtypeStruct((_K, _N // 2), jnp.int32),
        interpret=interpret,
    )


def _uv_body(s_ref, w5_ref, g5_ref, b5_ref, u_ref, v_ref):
    s = s_ref[...]
    wa = w5_ref[:, :_C]
    wd = w5_ref[:, _C:] - wa
    u = lax.dot_general(s, wa, (((0,), (1,)), ((), ())))
    v = lax.dot_general(s, wd, (((0,), (1,)), ((), ())))
    s5 = (g5_ref[0] * _BNS)[None, :]
    us = u * s5
    ub = lax.bitcast_convert_type(us, jnp.int32)
    ub = (ub + 0x7FFF + ((ub >> 16) & 1)) >> 16
    lo = ub[:, : _C // 2] & 0xFFFF
    hi = ub[:, _C // 2:] << 16
    u_ref[...] = lo | hi
    v_ref[...] = v * s5 + b5_ref[0][None, :]


def _make_uv(interpret=False):
    return pl.pallas_call(
        _uv_body,
        in_specs=[
            pl.BlockSpec((_C, _N), lambda: (0, 0)),
            pl.BlockSpec((_C, 2 * _C), lambda: (0, 0)),
            pl.BlockSpec((1, _C), lambda: (0, 0)),
            pl.BlockSpec((1, _C), lambda: (0, 0)),
        ],
        out_specs=[
            pl.BlockSpec((_N, _C // 2), lambda: (0, 0)),
            pl.BlockSpec((_N, _C), lambda: (0, 0)),
        ],
        out_shape=[
            jax.ShapeDtypeStruct((_N, _C // 2), jnp.int32),
            jax.ShapeDtypeStruct((_N, _C), jnp.float32),
        ],
        interpret=interpret,
    )


def _gather_body(tab_ref, idx_ref, out_ref, idx_v, rows_v, sem):
    wid = lax.axis_index("s") * 2 + lax.axis_index("c")
    base = wid * _PW

    def chunk(i, carry):
        off = base + i * _CR
        pltpu.sync_copy(idx_ref.at[pl.ds(off, _CR)], idx_v)
        pltpu.async_copy(tab_ref.at[idx_v], rows_v, sem).wait()
        pltpu.sync_copy(rows_v, out_ref.at[pl.ds(off, _CR)])
        return carry

    lax.fori_loop(0, _NCH, chunk, 0)


@functools.lru_cache(maxsize=None)
def _make_sc_gather():
    return pl.kernel(
        _gather_body,
        out_type=jax.ShapeDtypeStruct((_ROWS, _C // 2), jnp.int32),
        mesh=plsc.VectorSubcoreMesh(core_axis_name="c", subcore_axis_name="s"),
        scratch_types=[
            pltpu.VMEM((_CR,), jnp.int32),
            pltpu.VMEM((_CR, _C // 2), jnp.int32),
            pltpu.SemaphoreType.DMA,
        ],
    )


def _conv2_body(ga_ref, gb_ref, v_ref, w6_ref, g6_ref, b6_ref, out_ref):
    j = pl.program_id(0)
    gp = jnp.concatenate([ga_ref[...], gb_ref[...]], axis=0)
    g_lo = lax.bitcast_convert_type(gp << 16, jnp.float32)
    g_hi = lax.bitcast_convert_type(gp & jnp.int32(-65536), jnp.float32)
    h = jnp.concatenate([g_lo, g_hi], axis=1) + v_ref[...]
    h = jnp.maximum(h, 0.2 * h)
    f = lax.dot_general(w6_ref[...], h, (((1,), (0,)), ((), ())))
    a = f * (g6_ref[0] * _BNS)[:, None] + b6_ref[0][:, None]
    a = jnp.maximum(a, 0.2 * a)

    @pl.when(j == 0)
    def _():
        out_ref[...] = a

    @pl.when(j > 0)
    def _():
        out_ref[...] = jnp.maximum(out_ref[...], a)


def _make_conv2(interpret=False):
    return pl.pallas_call(
        _conv2_body,
        grid=(_K,),
        in_specs=[
            pl.BlockSpec((_N // 2, _C // 2), lambda j: (j, 0)),
            pl.BlockSpec((_N // 2, _C // 2), lambda j: (j, 0)),
            pl.BlockSpec((_N, _C), lambda j: (0, 0)),
            pl.BlockSpec((_O, _N), lambda j: (0, 0)),
            pl.BlockSpec((1, _O), lambda j: (0, 0)),
            pl.BlockSpec((1, _O), lambda j: (0, 0)),
        ],
        out_specs=pl.BlockSpec((_O, _C), lambda j: (0, 0)),
        out_shape=jax.ShapeDtypeStruct((_O, _C), jnp.float32),
        interpret=interpret,
    )


def _attn_body(xf_ref, yf_ref, wq_ref, wk_ref, wv_ref, ow_ref, ob_ref, out_ref):
    xf = xf_ref[...]
    yf = yf_ref[...]
    qt = lax.dot_general(wq_ref[...], xf, (((1,), (0,)), ((), ())))
    kt = lax.dot_general(wk_ref[...], yf, (((1,), (0,)), ((), ())))
    vt = lax.dot_general(wv_ref[...], yf, (((1,), (0,)), ((), ())))
    acc = jnp.zeros((_C, _O), jnp.float32)
    for h in range(_NH):
        sl = slice(h * _AD, (h + 1) * _AD)
        qh = qt[sl, :]
        kh = kt[sl, :]
        vh = vt[sl, :]
        dp = lax.dot_general(qh, kh, (((0,), (0,)), ((), ()))) * _SCALE
        m = jnp.max(dp, axis=1, keepdims=True)
        e = jnp.exp(dp - m)
        p = e / jnp.sum(e, axis=1, keepdims=True)
        wh = lax.dot_general(p, vh, (((1,), (1,)), ((), ())))
        owh = ow_ref[:, sl]
        acc = acc + lax.dot_general(wh, owh, (((1,), (1,)), ((), ())))
    out_ref[...] = acc + ob_ref[0][None, :]


def _make_attn(interpret=False):
    return pl.pallas_call(
        _attn_body,
        in_specs=[
            pl.BlockSpec((_O, _C), lambda: (0, 0)),
            pl.BlockSpec((_O, _C), lambda: (0, 0)),
            pl.BlockSpec((_NH * _AD, _O), lambda: (0, 0)),
            pl.BlockSpec((_NH * _AD, _O), lambda: (0, 0)),
            pl.BlockSpec((_NH * _AD, _O), lambda: (0, 0)),
            pl.BlockSpec((_O, _NH * _AD), lambda: (0, 0)),
            pl.BlockSpec((1, _O), lambda: (0, 0)),
        ],
        out_specs=pl.BlockSpec((_C, _O), lambda: (0, 0)),
        out_shape=jax.ShapeDtypeStruct((_C, _O), jnp.float32),
        interpret=interpret,
    )


def kernel(x, y, w5, g5, b5, w6, g6, b6, wq, wk, wv, ow, ob):
    sx = x[0]
    sy = y[0]
    g5r = g5.reshape(1, _C)
    b5r = b5.reshape(1, _C)
    g6r = g6.reshape(1, _O)
    b6r = b6.reshape(1, _O)
    obr = ob.reshape(1, _O)

    knn0 = _make_knn(0)
    knn1 = _make_knn(1)
    uv = _make_uv()
    conv2 = _make_conv2()
    gather = _make_sc_gather()

    ux, vx = uv(sx, w5, g5r, b5r)
    idx_xa = knn0(sx, sx)
    gxa = gather(ux, idx_xa.reshape(_ROWS))
    idx_xb = knn1(sx, sx)
    gxb = gather(ux, idx_xb.reshape(_ROWS))
    uy, vy = uv(sy, w5, g5r, b5r)
    idx_ya = knn0(sy, sy)
    gya = gather(uy, idx_ya.reshape(_ROWS))
    idx_yb = knn1(sy, sy)
    gyb = gather(uy, idx_yb.reshape(_ROWS))
    xft = conv2(gxa, gxb, vx, w6, g6r, b6r)
    yft = conv2(gya, gyb, vy, w6, g6r, b6r)
    out = _make_attn()(xft, yft, wq, wk, wv, ow, obr)
    return out[None]
